# baseline (speedup 1.0000x reference)
import jax
import jax.numpy as jnp
from jax.experimental import pallas as pl


def _copy_body(x_ref, o_ref):
    o_ref[...] = x_ref[...]


def kernel(locs, response, augmented_response, conditioning_sets, batch_idx):
    del response
    nn0 = conditioning_sets[1:, 0]
    diff = locs[1:, :] - jnp.take(locs, nn0, axis=0)
    scal = jnp.sqrt(jnp.sum(jnp.square(diff), axis=1))
    head = (jnp.square(scal[0]) / scal[4])[None]
    scal = jnp.concatenate([head, scal], axis=0)
    scal = scal / scal[0]
    locs_b = jnp.take(locs, batch_idx, axis=0)
    aug_b = jnp.take(augmented_response, batch_idx, axis=1)
    scales_b = jnp.take(scal, batch_idx, axis=0)
    locs_b = pl.pallas_call(
        _copy_body,
        out_shape=jax.ShapeDtypeStruct(locs_b.shape, locs_b.dtype),
    )(locs_b)
    return (locs_b, aug_b, scales_b)


# speedup vs baseline: 1.7759x; 1.7759x over previous
import functools

import jax
import jax.numpy as jnp
from jax import lax
from jax.experimental import pallas as pl
from jax.experimental.pallas import tpu as pltpu, tpu_sc as plsc

L = 16
NC = 2
NS = 16
NW = NC * NS
CH = 128
CA = 64


def _sqrt16(v):
    i = plsc.bitcast(v, jnp.int32)
    i = jnp.int32(0x5F3759DF) - (i >> 1)
    y = plsc.bitcast(i, jnp.float32)
    vh = v * jnp.float32(0.5)
    for _ in range(4):
        y = y * (jnp.float32(1.5) - vh * y * y)
    return v * y


def _dist2(rows_ref, n_chunks, lane):
    raise NotImplementedError


def _body(NREP, N, M1, B_PER_W, W,
          aug_hbm, loc_hbm, bidx_hbm,
          augout, lxout, lyout, scout,
          idx_v, hidx_v, hnn_v, hlA_v, hlB_v, d16_v,
          nn_v, lA_v, lB_v, scales_v, lx_v, ly_v, aug_v, gsem, osem):
    wid = lax.axis_index("s") * NC + lax.axis_index("c")
    base = wid * B_PER_W
    lane = lax.iota(jnp.int32, L)
    c0 = jnp.zeros((L,), jnp.int32)
    c1 = jnp.full((L,), 1, jnp.int32)
    c2 = jnp.full((L,), 2, jnp.int32)

    pltpu.sync_copy(bidx_hbm.at[pl.ds(base, B_PER_W)], idx_v)

    hidx = jnp.where(lane == 1, 5, jnp.where(lane == 0, 1, lane + L + wid * L))
    hidx_v[...] = hidx
    pltpu.sync_copy(loc_hbm.at[hidx_v], hlA_v)
    hnn_v[...] = plsc.load_gather(hlA_v, [lane, c2]).astype(jnp.int32)
    pltpu.sync_copy(loc_hbm.at[hnn_v], hlB_v)
    dx = plsc.load_gather(hlA_v, [lane, c0]) - plsc.load_gather(hlB_v, [lane, c0])
    dy = plsc.load_gather(hlA_v, [lane, c1]) - plsc.load_gather(hlB_v, [lane, c1])
    d16_v[...] = dx * dx + dy * dy
    d16 = d16_v[...]
    d1 = d16[0]
    d5 = d16[1]

    for ci in range(B_PER_W // CH):
        idx_sl = idx_v.at[pl.ds(ci * CH, CH)]
        pltpu.sync_copy(loc_hbm.at[idx_sl], lA_v)
        for j in range(CH // L):
            rows = lane + j * L
            nn_v[pl.ds(j * L, L)] = plsc.load_gather(
                lA_v, [rows, c2]).astype(jnp.int32)
        pltpu.sync_copy(loc_hbm.at[nn_v], lB_v)
        for j in range(CH // L):
            rows = lane + j * L
            ax = plsc.load_gather(lA_v, [rows, c0])
            ay = plsc.load_gather(lA_v, [rows, c1])
            dx = ax - plsc.load_gather(lB_v, [rows, c0])
            dy = ay - plsc.load_gather(lB_v, [rows, c1])
            d = dx * dx + dy * dy
            sc = _sqrt16(d * d5) / d1
            bi = idx_v[pl.ds(ci * CH + j * L, L)]
            scales_v[pl.ds(j * L, L)] = jnp.where(bi == 0, jnp.float32(1.0), sc)
            lx_v[pl.ds(j * L, L)] = ax
            ly_v[pl.ds(j * L, L)] = ay
        pltpu.sync_copy(scales_v, scout.at[pl.ds(base + ci * CH, CH)])
        pltpu.sync_copy(lx_v, lxout.at[pl.ds(base + ci * CH, CH)])
        pltpu.sync_copy(ly_v, lyout.at[pl.ds(base + ci * CH, CH)])

    nwin = B_PER_W // CA
    gathers = [None, None]
    writes = [None, None]
    for t in range(nwin):
        b = t % 2
        if writes[b] is not None:
            writes[b].wait()
        g = pltpu.make_async_copy(
            aug_hbm.at[idx_v.at[pl.ds(t * CA, CA)]], aug_v.at[b], gsem.at[b])
        g.start()
        gathers[b] = g
        if t >= 1:
            pb = 1 - b
            gathers[pb].wait()
            w = pltpu.make_async_copy(
                aug_v.at[pb], augout.at[pl.ds(base + (t - 1) * CA, CA)],
                osem.at[pb])
            w.start()
            writes[pb] = w
    lb = (nwin - 1) % 2
    gathers[lb].wait()
    w = pltpu.make_async_copy(
        aug_v.at[lb], augout.at[pl.ds(base + (nwin - 1) * CA, CA)], osem.at[lb])
    w.start()
    writes[lb] = w
    writes[0].wait()
    writes[1].wait()


def kernel(locs, response, augmented_response, conditioning_sets, batch_idx):
    del response
    N, _ = locs.shape
    NREP, _, M1 = augmented_response.shape
    B = batch_idx.shape[0]
    B_PER_W = B // NW
    W = 512

    aug_rows = augmented_response.transpose(1, 0, 2).reshape(N, NREP * M1)
    aug_p = jnp.pad(aug_rows, ((0, 0), (0, W - NREP * M1)))
    cs0f = conditioning_sets[:, :1].astype(jnp.float32)
    loc128 = jnp.concatenate(
        [locs, cs0f, jnp.zeros((N, 128 - 3), jnp.float32)], axis=1)

    mesh = plsc.VectorSubcoreMesh(core_axis_name="c", subcore_axis_name="s")
    body = functools.partial(_body, NREP, N, M1, B_PER_W, W)
    augout, lx, ly, scales_b = pl.kernel(
        body,
        mesh=mesh,
        compiler_params=pltpu.CompilerParams(needs_layout_passes=False),
        out_type=[
            jax.ShapeDtypeStruct((B, W), jnp.float32),
            jax.ShapeDtypeStruct((B,), jnp.float32),
            jax.ShapeDtypeStruct((B,), jnp.float32),
            jax.ShapeDtypeStruct((B,), jnp.float32),
        ],
        scratch_types=[
            pltpu.VMEM((B_PER_W,), jnp.int32),
            pltpu.VMEM((L,), jnp.int32),
            pltpu.VMEM((L,), jnp.int32),
            pltpu.VMEM((L, 128), jnp.float32),
            pltpu.VMEM((L, 128), jnp.float32),
            pltpu.VMEM((L,), jnp.float32),
            pltpu.VMEM((CH,), jnp.int32),
            pltpu.VMEM((CH, 128), jnp.float32),
            pltpu.VMEM((CH, 128), jnp.float32),
            pltpu.VMEM((CH,), jnp.float32),
            pltpu.VMEM((CH,), jnp.float32),
            pltpu.VMEM((CH,), jnp.float32),
            pltpu.VMEM((2, CA, W), jnp.float32),
            pltpu.SemaphoreType.DMA((2,)),
            pltpu.SemaphoreType.DMA((2,)),
        ],
    )(aug_p, loc128, batch_idx)

    locs_b = jnp.stack([lx, ly], axis=1)
    aug_b = augout[:, :NREP * M1].reshape(B, NREP, M1).transpose(1, 0, 2)
    return locs_b, aug_b, scales_b


# speedup vs baseline: 3.3721x; 1.8988x over previous
import functools

import jax
import jax.numpy as jnp
from jax import lax
from jax.experimental import pallas as pl
from jax.experimental.pallas import tpu as pltpu, tpu_sc as plsc

L = 16
NC = 2
NS = 16
NW = NC * NS
CH = 128
CA = 64


def _sqrt16(v):
    i = plsc.bitcast(v, jnp.int32)
    i = jnp.int32(0x5F3759DF) - (i >> 1)
    y = plsc.bitcast(i, jnp.float32)
    vh = v * jnp.float32(0.5)
    for _ in range(4):
        y = y * (jnp.float32(1.5) - vh * y * y)
    return v * y


def _build_body(in_ref, out_ref):
    x = in_ref[...].reshape(496, 128)
    out_ref[:, :496] = jnp.transpose(x)
    out_ref[:, 496:] = jnp.zeros((128, 16), jnp.float32)


def _unpack_body(in_ref, out_ref):
    x = in_ref[...]
    out_ref[...] = jnp.transpose(x[:, :496]).reshape(31, 16, 128)


def _body(NREP, N, M1, B_PER_W, W,
          aug_hbm, loc_hbm, bidx_hbm,
          augout, lxout, lyout, scout,
          idx_v, hidx_v, hnn_v, hlA_v, hlB_v, d16_v,
          nn_v, lA_v, lB_v, scales_v, lx_v, ly_v, aug_v, gsem, osem):
    wid = lax.axis_index("s") * NC + lax.axis_index("c")
    base = wid * B_PER_W
    lane = lax.iota(jnp.int32, L)
    c0 = jnp.zeros((L,), jnp.int32)
    c1 = jnp.full((L,), 1, jnp.int32)
    c2 = jnp.full((L,), 2, jnp.int32)

    pltpu.sync_copy(bidx_hbm.at[pl.ds(base, B_PER_W)], idx_v)

    hidx = jnp.where(lane == 1, 5, jnp.where(lane == 0, 1, lane + L + wid * L))
    hidx_v[...] = hidx
    pltpu.sync_copy(loc_hbm.at[hidx_v], hlA_v)
    hnn_v[...] = plsc.load_gather(hlA_v, [lane, c2]).astype(jnp.int32)
    pltpu.sync_copy(loc_hbm.at[hnn_v], hlB_v)
    dx = plsc.load_gather(hlA_v, [lane, c0]) - plsc.load_gather(hlB_v, [lane, c0])
    dy = plsc.load_gather(hlA_v, [lane, c1]) - plsc.load_gather(hlB_v, [lane, c1])
    d16_v[...] = dx * dx + dy * dy
    d16 = d16_v[...]
    d1 = d16[0]
    d5 = d16[1]

    for ci in range(B_PER_W // CH):
        idx_sl = idx_v.at[pl.ds(ci * CH, CH)]
        pltpu.sync_copy(loc_hbm.at[idx_sl], lA_v)
        for j in range(CH // L):
            rows = lane + j * L
            nn_v[pl.ds(j * L, L)] = plsc.load_gather(
                lA_v, [rows, c2]).astype(jnp.int32)
        pltpu.sync_copy(loc_hbm.at[nn_v], lB_v)
        for j in range(CH // L):
            rows = lane + j * L
            ax = plsc.load_gather(lA_v, [rows, c0])
            ay = plsc.load_gather(lA_v, [rows, c1])
            dx = ax - plsc.load_gather(lB_v, [rows, c0])
            dy = ay - plsc.load_gather(lB_v, [rows, c1])
            d = dx * dx + dy * dy
            sc = _sqrt16(d * d5) / d1
            bi = idx_v[pl.ds(ci * CH + j * L, L)]
            scales_v[pl.ds(j * L, L)] = jnp.where(bi == 0, jnp.float32(1.0), sc)
            lx_v[pl.ds(j * L, L)] = ax
            ly_v[pl.ds(j * L, L)] = ay
        pltpu.sync_copy(scales_v, scout.at[pl.ds(base + ci * CH, CH)])
        pltpu.sync_copy(lx_v, lxout.at[pl.ds(base + ci * CH, CH)])
        pltpu.sync_copy(ly_v, lyout.at[pl.ds(base + ci * CH, CH)])

    nwin = B_PER_W // CA
    gathers = [None, None]
    writes = [None, None]
    for t in range(nwin):
        b = t % 2
        if writes[b] is not None:
            writes[b].wait()
        g = pltpu.make_async_copy(
            aug_hbm.at[idx_v.at[pl.ds(t * CA, CA)]], aug_v.at[b], gsem.at[b])
        g.start()
        gathers[b] = g
        if t >= 1:
            pb = 1 - b
            gathers[pb].wait()
            w = pltpu.make_async_copy(
                aug_v.at[pb], augout.at[pl.ds(base + (t - 1) * CA, CA)],
                osem.at[pb])
            w.start()
            writes[pb] = w
    lb = (nwin - 1) % 2
    gathers[lb].wait()
    w = pltpu.make_async_copy(
        aug_v.at[lb], augout.at[pl.ds(base + (nwin - 1) * CA, CA)], osem.at[lb])
    w.start()
    writes[lb] = w
    writes[0].wait()
    writes[1].wait()


def kernel(locs, response, augmented_response, conditioning_sets, batch_idx):
    del response
    N, _ = locs.shape
    NREP, _, M1 = augmented_response.shape
    B = batch_idx.shape[0]
    B_PER_W = B // NW
    W = 512

    aug_t = augmented_response.transpose(2, 0, 1)
    aug_p = pl.pallas_call(
        _build_body,
        grid=(pl.cdiv(N, 128),),
        in_specs=[pl.BlockSpec((M1, NREP, 128), lambda i: (0, 0, i))],
        out_specs=pl.BlockSpec((128, W), lambda i: (i, 0)),
        out_shape=jax.ShapeDtypeStruct((N, W), jnp.float32),
    )(aug_t)
    cs0f = conditioning_sets[:, :1].astype(jnp.float32)
    loc128 = jnp.concatenate(
        [locs, cs0f, jnp.zeros((N, 128 - 3), jnp.float32)], axis=1)

    mesh = plsc.VectorSubcoreMesh(core_axis_name="c", subcore_axis_name="s")
    body = functools.partial(_body, NREP, N, M1, B_PER_W, W)
    augout, lx, ly, scales_b = pl.kernel(
        body,
        mesh=mesh,
        compiler_params=pltpu.CompilerParams(needs_layout_passes=False),
        out_type=[
            jax.ShapeDtypeStruct((B, W), jnp.float32),
            jax.ShapeDtypeStruct((B,), jnp.float32),
            jax.ShapeDtypeStruct((B,), jnp.float32),
            jax.ShapeDtypeStruct((B,), jnp.float32),
        ],
        scratch_types=[
            pltpu.VMEM((B_PER_W,), jnp.int32),
            pltpu.VMEM((L,), jnp.int32),
            pltpu.VMEM((L,), jnp.int32),
            pltpu.VMEM((L, 128), jnp.float32),
            pltpu.VMEM((L, 128), jnp.float32),
            pltpu.VMEM((L,), jnp.float32),
            pltpu.VMEM((CH,), jnp.int32),
            pltpu.VMEM((CH, 128), jnp.float32),
            pltpu.VMEM((CH, 128), jnp.float32),
            pltpu.VMEM((CH,), jnp.float32),
            pltpu.VMEM((CH,), jnp.float32),
            pltpu.VMEM((CH,), jnp.float32),
            pltpu.VMEM((2, CA, W), jnp.float32),
            pltpu.SemaphoreType.DMA((2,)),
            pltpu.SemaphoreType.DMA((2,)),
        ],
    )(aug_p, loc128, batch_idx)

    locs_b = jnp.stack([lx, ly], axis=1)
    aug_n = pl.pallas_call(
        _unpack_body,
        grid=(B // 128,),
        in_specs=[pl.BlockSpec((128, W), lambda i: (i, 0))],
        out_specs=pl.BlockSpec((M1, NREP, 128), lambda i: (0, 0, i)),
        out_shape=jax.ShapeDtypeStruct((M1, NREP, B), jnp.float32),
    )(augout)
    aug_b = aug_n.transpose(1, 2, 0)
    return locs_b, aug_b, scales_b


# speedup vs baseline: 5.9937x; 1.7774x over previous
import functools

import jax
import jax.numpy as jnp
from jax import lax
from jax.experimental import pallas as pl
from jax.experimental.pallas import tpu as pltpu, tpu_sc as plsc

L = 16
NC = 2
NS = 16
NW = NC * NS
CH = 128
CA = 64


def _sqrt16(v):
    i = plsc.bitcast(v, jnp.int32)
    i = jnp.int32(0x5F3759DF) - (i >> 1)
    y = plsc.bitcast(i, jnp.float32)
    vh = v * jnp.float32(0.5)
    for _ in range(4):
        y = y * (jnp.float32(1.5) - vh * y * y)
    return v * y


BN = 512


def _build_body(in_ref, out_ref):
    x = in_ref[...].reshape(496, BN)
    out_ref[:, :496] = jnp.transpose(x)
    out_ref[:, 496:] = jnp.zeros((BN, 16), jnp.float32)


def _unpack_body(in_ref, out_ref):
    x = in_ref[...]
    out_ref[...] = jnp.transpose(x[:, :496]).reshape(31, 16, BN)


def _body(NREP, N, M1, B_PER_W, W,
          aug_hbm, loc_hbm, bidx_hbm,
          augout, lxout, lyout, scout,
          idx_v, hidx_v, hnn_v, hlA_v, hlB_v, d16_v,
          nn_v, lA_v, lB_v, scales_v, lx_v, ly_v, aug_v, gsem, osem):
    wid = lax.axis_index("s") * NC + lax.axis_index("c")
    base = wid * B_PER_W
    lane = lax.iota(jnp.int32, L)
    c0 = jnp.zeros((L,), jnp.int32)
    c1 = jnp.full((L,), 1, jnp.int32)
    c2 = jnp.full((L,), 2, jnp.int32)

    pltpu.sync_copy(bidx_hbm.at[pl.ds(base, B_PER_W)], idx_v)

    hidx = jnp.where(lane == 1, 5, jnp.where(lane == 0, 1, lane + L + wid * L))
    hidx_v[...] = hidx
    pltpu.sync_copy(loc_hbm.at[hidx_v], hlA_v)
    hnn_v[...] = plsc.load_gather(hlA_v, [lane, c2]).astype(jnp.int32)
    pltpu.sync_copy(loc_hbm.at[hnn_v], hlB_v)
    dx = plsc.load_gather(hlA_v, [lane, c0]) - plsc.load_gather(hlB_v, [lane, c0])
    dy = plsc.load_gather(hlA_v, [lane, c1]) - plsc.load_gather(hlB_v, [lane, c1])
    d16_v[...] = dx * dx + dy * dy
    d16 = d16_v[...]
    d1 = d16[0]
    d5 = d16[1]

    for ci in range(B_PER_W // CH):
        idx_sl = idx_v.at[pl.ds(ci * CH, CH)]
        pltpu.sync_copy(loc_hbm.at[idx_sl], lA_v)
        for j in range(CH // L):
            rows = lane + j * L
            nn_v[pl.ds(j * L, L)] = plsc.load_gather(
                lA_v, [rows, c2]).astype(jnp.int32)
        pltpu.sync_copy(loc_hbm.at[nn_v], lB_v)
        for j in range(CH // L):
            rows = lane + j * L
            ax = plsc.load_gather(lA_v, [rows, c0])
            ay = plsc.load_gather(lA_v, [rows, c1])
            dx = ax - plsc.load_gather(lB_v, [rows, c0])
            dy = ay - plsc.load_gather(lB_v, [rows, c1])
            d = dx * dx + dy * dy
            sc = _sqrt16(d * d5) / d1
            bi = idx_v[pl.ds(ci * CH + j * L, L)]
            scales_v[pl.ds(j * L, L)] = jnp.where(bi == 0, jnp.float32(1.0), sc)
            lx_v[pl.ds(j * L, L)] = ax
            ly_v[pl.ds(j * L, L)] = ay
        pltpu.sync_copy(scales_v, scout.at[pl.ds(base + ci * CH, CH)])
        pltpu.sync_copy(lx_v, lxout.at[pl.ds(base + ci * CH, CH)])
        pltpu.sync_copy(ly_v, lyout.at[pl.ds(base + ci * CH, CH)])

    nwin = B_PER_W // CA
    gathers = [None, None]
    writes = [None, None]
    for t in range(nwin):
        b = t % 2
        if writes[b] is not None:
            writes[b].wait()
        g = pltpu.make_async_copy(
            aug_hbm.at[idx_v.at[pl.ds(t * CA, CA)]], aug_v.at[b], gsem.at[b])
        g.start()
        gathers[b] = g
        if t >= 1:
            pb = 1 - b
            gathers[pb].wait()
            w = pltpu.make_async_copy(
                aug_v.at[pb], augout.at[pl.ds(base + (t - 1) * CA, CA)],
                osem.at[pb])
            w.start()
            writes[pb] = w
    lb = (nwin - 1) % 2
    gathers[lb].wait()
    w = pltpu.make_async_copy(
        aug_v.at[lb], augout.at[pl.ds(base + (nwin - 1) * CA, CA)], osem.at[lb])
    w.start()
    writes[lb] = w
    writes[0].wait()
    writes[1].wait()


def kernel(locs, response, augmented_response, conditioning_sets, batch_idx):
    del response
    N, _ = locs.shape
    NREP, _, M1 = augmented_response.shape
    B = batch_idx.shape[0]
    B_PER_W = B // NW
    W = 512

    aug_t = augmented_response.transpose(2, 0, 1)
    aug_p = pl.pallas_call(
        _build_body,
        grid=(pl.cdiv(N, BN),),
        in_specs=[pl.BlockSpec((M1, NREP, BN), lambda i: (0, 0, i))],
        out_specs=pl.BlockSpec((BN, W), lambda i: (i, 0)),
        out_shape=jax.ShapeDtypeStruct((N, W), jnp.float32),
    )(aug_t)
    cs0f = conditioning_sets[:, :1].astype(jnp.float32)
    loc128 = jnp.concatenate(
        [locs, cs0f, jnp.zeros((N, 128 - 3), jnp.float32)], axis=1)

    mesh = plsc.VectorSubcoreMesh(core_axis_name="c", subcore_axis_name="s")
    body = functools.partial(_body, NREP, N, M1, B_PER_W, W)
    augout, lx, ly, scales_b = pl.kernel(
        body,
        mesh=mesh,
        compiler_params=pltpu.CompilerParams(needs_layout_passes=False),
        out_type=[
            jax.ShapeDtypeStruct((B, W), jnp.float32),
            jax.ShapeDtypeStruct((B,), jnp.float32),
            jax.ShapeDtypeStruct((B,), jnp.float32),
            jax.ShapeDtypeStruct((B,), jnp.float32),
        ],
        scratch_types=[
            pltpu.VMEM((B_PER_W,), jnp.int32),
            pltpu.VMEM((L,), jnp.int32),
            pltpu.VMEM((L,), jnp.int32),
            pltpu.VMEM((L, 128), jnp.float32),
            pltpu.VMEM((L, 128), jnp.float32),
            pltpu.VMEM((L,), jnp.float32),
            pltpu.VMEM((CH,), jnp.int32),
            pltpu.VMEM((CH, 128), jnp.float32),
            pltpu.VMEM((CH, 128), jnp.float32),
            pltpu.VMEM((CH,), jnp.float32),
            pltpu.VMEM((CH,), jnp.float32),
            pltpu.VMEM((CH,), jnp.float32),
            pltpu.VMEM((2, CA, W), jnp.float32),
            pltpu.SemaphoreType.DMA((2,)),
            pltpu.SemaphoreType.DMA((2,)),
        ],
    )(aug_p, loc128, batch_idx)

    locs_b = jnp.stack([lx, ly], axis=1)
    aug_n = pl.pallas_call(
        _unpack_body,
        grid=(B // BN,),
        in_specs=[pl.BlockSpec((BN, W), lambda i: (i, 0))],
        out_specs=pl.BlockSpec((M1, NREP, BN), lambda i: (0, 0, i)),
        out_shape=jax.ShapeDtypeStruct((M1, NREP, B), jnp.float32),
    )(augout)
    aug_b = aug_n.transpose(1, 2, 0)
    return locs_b, aug_b, scales_b


# speedup vs baseline: 7.6199x; 1.2713x over previous
import functools

import jax
import jax.numpy as jnp
from jax import lax
from jax.experimental import pallas as pl
from jax.experimental.pallas import tpu as pltpu, tpu_sc as plsc

L = 16
NC = 2
NS = 16
NW = NC * NS
CH = 128
CA = 64


def _sqrt16(v):
    i = plsc.bitcast(v, jnp.int32)
    i = jnp.int32(0x5F3759DF) - (i >> 1)
    y = plsc.bitcast(i, jnp.float32)
    vh = v * jnp.float32(0.5)
    for _ in range(4):
        y = y * (jnp.float32(1.5) - vh * y * y)
    return v * y


BN = 2048


def _build_body(in_ref, out_ref):
    x = in_ref[...].reshape(496, BN)
    out_ref[:, :496] = jnp.transpose(x)
    out_ref[:, 496:] = jnp.zeros((BN, 16), jnp.float32)


def _unpack_body(in_ref, out_ref):
    x = in_ref[...]
    out_ref[...] = jnp.transpose(x[:, :496]).reshape(31, 16, BN)


def _body(NREP, N, M1, B_PER_W, W,
          aug_hbm, loc_hbm, bidx_hbm,
          augout, lxout, lyout, scout,
          idx_v, hidx_v, hnn_v, hlA_v, hlB_v, d16_v,
          nn_v, lA_v, lB_v, scales_v, lx_v, ly_v, aug_v, gsem, osem):
    wid = lax.axis_index("s") * NC + lax.axis_index("c")
    base = wid * B_PER_W
    lane = lax.iota(jnp.int32, L)
    c0 = jnp.zeros((L,), jnp.int32)
    c1 = jnp.full((L,), 1, jnp.int32)
    c2 = jnp.full((L,), 2, jnp.int32)

    pltpu.sync_copy(bidx_hbm.at[pl.ds(base, B_PER_W)], idx_v)

    hidx = jnp.where(lane == 1, 5, jnp.where(lane == 0, 1, lane + L + wid * L))
    hidx_v[...] = hidx
    pltpu.sync_copy(loc_hbm.at[hidx_v], hlA_v)
    hnn_v[...] = plsc.load_gather(hlA_v, [lane, c2]).astype(jnp.int32)
    pltpu.sync_copy(loc_hbm.at[hnn_v], hlB_v)
    dx = plsc.load_gather(hlA_v, [lane, c0]) - plsc.load_gather(hlB_v, [lane, c0])
    dy = plsc.load_gather(hlA_v, [lane, c1]) - plsc.load_gather(hlB_v, [lane, c1])
    d16_v[...] = dx * dx + dy * dy
    d16 = d16_v[...]
    d1 = d16[0]
    d5 = d16[1]

    for ci in range(B_PER_W // CH):
        idx_sl = idx_v.at[pl.ds(ci * CH, CH)]
        pltpu.sync_copy(loc_hbm.at[idx_sl], lA_v)
        for j in range(CH // L):
            rows = lane + j * L
            nn_v[pl.ds(j * L, L)] = plsc.load_gather(
                lA_v, [rows, c2]).astype(jnp.int32)
        pltpu.sync_copy(loc_hbm.at[nn_v], lB_v)
        for j in range(CH // L):
            rows = lane + j * L
            ax = plsc.load_gather(lA_v, [rows, c0])
            ay = plsc.load_gather(lA_v, [rows, c1])
            dx = ax - plsc.load_gather(lB_v, [rows, c0])
            dy = ay - plsc.load_gather(lB_v, [rows, c1])
            d = dx * dx + dy * dy
            sc = _sqrt16(d * d5) / d1
            bi = idx_v[pl.ds(ci * CH + j * L, L)]
            scales_v[pl.ds(j * L, L)] = jnp.where(bi == 0, jnp.float32(1.0), sc)
            lx_v[pl.ds(j * L, L)] = ax
            ly_v[pl.ds(j * L, L)] = ay
        pltpu.sync_copy(scales_v, scout.at[pl.ds(base + ci * CH, CH)])
        pltpu.sync_copy(lx_v, lxout.at[pl.ds(base + ci * CH, CH)])
        pltpu.sync_copy(ly_v, lyout.at[pl.ds(base + ci * CH, CH)])

    nwin = B_PER_W // CA
    gathers = [None, None]
    writes = [None, None]
    for t in range(nwin):
        b = t % 2
        if writes[b] is not None:
            writes[b].wait()
        g = pltpu.make_async_copy(
            aug_hbm.at[idx_v.at[pl.ds(t * CA, CA)]], aug_v.at[b], gsem.at[b])
        g.start()
        gathers[b] = g
        if t >= 1:
            pb = 1 - b
            gathers[pb].wait()
            w = pltpu.make_async_copy(
                aug_v.at[pb], augout.at[pl.ds(base + (t - 1) * CA, CA)],
                osem.at[pb])
            w.start()
            writes[pb] = w
    lb = (nwin - 1) % 2
    gathers[lb].wait()
    w = pltpu.make_async_copy(
        aug_v.at[lb], augout.at[pl.ds(base + (nwin - 1) * CA, CA)], osem.at[lb])
    w.start()
    writes[lb] = w
    writes[0].wait()
    writes[1].wait()


def kernel(locs, response, augmented_response, conditioning_sets, batch_idx):
    del response
    N, _ = locs.shape
    NREP, _, M1 = augmented_response.shape
    B = batch_idx.shape[0]
    B_PER_W = B // NW
    W = 512

    aug_t = augmented_response.transpose(2, 0, 1)
    aug_p = pl.pallas_call(
        _build_body,
        grid=(pl.cdiv(N, BN),),
        in_specs=[pl.BlockSpec((M1, NREP, BN), lambda i: (0, 0, i))],
        out_specs=pl.BlockSpec((BN, W), lambda i: (i, 0)),
        out_shape=jax.ShapeDtypeStruct((N, W), jnp.float32),
    )(aug_t)
    cs0f = conditioning_sets[:, :1].astype(jnp.float32)
    loc128 = jnp.concatenate(
        [locs, cs0f, jnp.zeros((N, 128 - 3), jnp.float32)], axis=1)

    mesh = plsc.VectorSubcoreMesh(core_axis_name="c", subcore_axis_name="s")
    body = functools.partial(_body, NREP, N, M1, B_PER_W, W)
    augout, lx, ly, scales_b = pl.kernel(
        body,
        mesh=mesh,
        compiler_params=pltpu.CompilerParams(needs_layout_passes=False),
        out_type=[
            jax.ShapeDtypeStruct((B, W), jnp.float32),
            jax.ShapeDtypeStruct((B,), jnp.float32),
            jax.ShapeDtypeStruct((B,), jnp.float32),
            jax.ShapeDtypeStruct((B,), jnp.float32),
        ],
        scratch_types=[
            pltpu.VMEM((B_PER_W,), jnp.int32),
            pltpu.VMEM((L,), jnp.int32),
            pltpu.VMEM((L,), jnp.int32),
            pltpu.VMEM((L, 128), jnp.float32),
            pltpu.VMEM((L, 128), jnp.float32),
            pltpu.VMEM((L,), jnp.float32),
            pltpu.VMEM((CH,), jnp.int32),
            pltpu.VMEM((CH, 128), jnp.float32),
            pltpu.VMEM((CH, 128), jnp.float32),
            pltpu.VMEM((CH,), jnp.float32),
            pltpu.VMEM((CH,), jnp.float32),
            pltpu.VMEM((CH,), jnp.float32),
            pltpu.VMEM((2, CA, W), jnp.float32),
            pltpu.SemaphoreType.DMA((2,)),
            pltpu.SemaphoreType.DMA((2,)),
        ],
    )(aug_p, loc128, batch_idx)

    locs_b = jnp.stack([lx, ly], axis=1)
    aug_n = pl.pallas_call(
        _unpack_body,
        grid=(B // BN,),
        in_specs=[pl.BlockSpec((BN, W), lambda i: (i, 0))],
        out_specs=pl.BlockSpec((M1, NREP, BN), lambda i: (0, 0, i)),
        out_shape=jax.ShapeDtypeStruct((M1, NREP, B), jnp.float32),
    )(augout)
    aug_b = aug_n.transpose(1, 2, 0)
    return locs_b, aug_b, scales_b


# speedup vs baseline: 7.8112x; 1.0251x over previous
import functools

import jax
import jax.numpy as jnp
from jax import lax
from jax.experimental import pallas as pl
from jax.experimental.pallas import tpu as pltpu, tpu_sc as plsc

L = 16
NC = 2
NS = 16
NW = NC * NS
CH = 128
CA = 64


def _sqrt16(v):
    i = plsc.bitcast(v, jnp.int32)
    i = jnp.int32(0x5F3759DF) - (i >> 1)
    y = plsc.bitcast(i, jnp.float32)
    vh = v * jnp.float32(0.5)
    for _ in range(4):
        y = y * (jnp.float32(1.5) - vh * y * y)
    return v * y


BN = 4096


def _build_body(in_ref, out_ref):
    x = in_ref[...].reshape(496, BN)
    out_ref[:, :496] = jnp.transpose(x)
    out_ref[:, 496:] = jnp.zeros((BN, 16), jnp.float32)


def _unpack_body(in_ref, out_ref):
    x = in_ref[...]
    out_ref[...] = jnp.transpose(x[:, :496]).reshape(31, 16, BN)


def _body(NREP, N, M1, B_PER_W, W,
          aug_hbm, loc_hbm, bidx_hbm,
          augout, lxout, lyout, scout,
          idx_v, hidx_v, hnn_v, hlA_v, hlB_v, d16_v,
          nn_v, lA_v, lB_v, scales_v, lx_v, ly_v, aug_v, gsem, osem):
    wid = lax.axis_index("s") * NC + lax.axis_index("c")
    base = wid * B_PER_W
    lane = lax.iota(jnp.int32, L)
    c0 = jnp.zeros((L,), jnp.int32)
    c1 = jnp.full((L,), 1, jnp.int32)
    c2 = jnp.full((L,), 2, jnp.int32)

    pltpu.sync_copy(bidx_hbm.at[pl.ds(base, B_PER_W)], idx_v)

    hidx = jnp.where(lane == 1, 5, jnp.where(lane == 0, 1, lane + L + wid * L))
    hidx_v[...] = hidx
    pltpu.sync_copy(loc_hbm.at[hidx_v], hlA_v)
    hnn_v[...] = plsc.load_gather(hlA_v, [lane, c2]).astype(jnp.int32)
    pltpu.sync_copy(loc_hbm.at[hnn_v], hlB_v)
    dx = plsc.load_gather(hlA_v, [lane, c0]) - plsc.load_gather(hlB_v, [lane, c0])
    dy = plsc.load_gather(hlA_v, [lane, c1]) - plsc.load_gather(hlB_v, [lane, c1])
    d16_v[...] = dx * dx + dy * dy
    d16 = d16_v[...]
    d1 = d16[0]
    d5 = d16[1]

    for ci in range(B_PER_W // CH):
        idx_sl = idx_v.at[pl.ds(ci * CH, CH)]
        pltpu.sync_copy(loc_hbm.at[idx_sl], lA_v)
        for j in range(CH // L):
            rows = lane + j * L
            nn_v[pl.ds(j * L, L)] = plsc.load_gather(
                lA_v, [rows, c2]).astype(jnp.int32)
        pltpu.sync_copy(loc_hbm.at[nn_v], lB_v)
        for j in range(CH // L):
            rows = lane + j * L
            ax = plsc.load_gather(lA_v, [rows, c0])
            ay = plsc.load_gather(lA_v, [rows, c1])
            dx = ax - plsc.load_gather(lB_v, [rows, c0])
            dy = ay - plsc.load_gather(lB_v, [rows, c1])
            d = dx * dx + dy * dy
            sc = _sqrt16(d * d5) / d1
            bi = idx_v[pl.ds(ci * CH + j * L, L)]
            scales_v[pl.ds(j * L, L)] = jnp.where(bi == 0, jnp.float32(1.0), sc)
            lx_v[pl.ds(j * L, L)] = ax
            ly_v[pl.ds(j * L, L)] = ay
        pltpu.sync_copy(scales_v, scout.at[pl.ds(base + ci * CH, CH)])
        pltpu.sync_copy(lx_v, lxout.at[pl.ds(base + ci * CH, CH)])
        pltpu.sync_copy(ly_v, lyout.at[pl.ds(base + ci * CH, CH)])

    nwin = B_PER_W // CA
    gathers = [None, None]
    writes = [None, None]
    for t in range(nwin):
        b = t % 2
        if writes[b] is not None:
            writes[b].wait()
        g = pltpu.make_async_copy(
            aug_hbm.at[idx_v.at[pl.ds(t * CA, CA)]], aug_v.at[b], gsem.at[b])
        g.start()
        gathers[b] = g
        if t >= 1:
            pb = 1 - b
            gathers[pb].wait()
            w = pltpu.make_async_copy(
                aug_v.at[pb], augout.at[pl.ds(base + (t - 1) * CA, CA)],
                osem.at[pb])
            w.start()
            writes[pb] = w
    lb = (nwin - 1) % 2
    gathers[lb].wait()
    w = pltpu.make_async_copy(
        aug_v.at[lb], augout.at[pl.ds(base + (nwin - 1) * CA, CA)], osem.at[lb])
    w.start()
    writes[lb] = w
    writes[0].wait()
    writes[1].wait()


def kernel(locs, response, augmented_response, conditioning_sets, batch_idx):
    del response
    N, _ = locs.shape
    NREP, _, M1 = augmented_response.shape
    B = batch_idx.shape[0]
    B_PER_W = B // NW
    W = 512

    aug_t = augmented_response.transpose(2, 0, 1)
    aug_p = pl.pallas_call(
        _build_body,
        grid=(pl.cdiv(N, BN),),
        in_specs=[pl.BlockSpec((M1, NREP, BN), lambda i: (0, 0, i))],
        out_specs=pl.BlockSpec((BN, W), lambda i: (i, 0)),
        out_shape=jax.ShapeDtypeStruct((N, W), jnp.float32),
    )(aug_t)
    cs0f = conditioning_sets[:, :1].astype(jnp.float32)
    loc128 = jnp.concatenate(
        [locs, cs0f, jnp.zeros((N, 128 - 3), jnp.float32)], axis=1)

    mesh = plsc.VectorSubcoreMesh(core_axis_name="c", subcore_axis_name="s")
    body = functools.partial(_body, NREP, N, M1, B_PER_W, W)
    augout, lx, ly, scales_b = pl.kernel(
        body,
        mesh=mesh,
        compiler_params=pltpu.CompilerParams(needs_layout_passes=False),
        out_type=[
            jax.ShapeDtypeStruct((B, W), jnp.float32),
            jax.ShapeDtypeStruct((B,), jnp.float32),
            jax.ShapeDtypeStruct((B,), jnp.float32),
            jax.ShapeDtypeStruct((B,), jnp.float32),
        ],
        scratch_types=[
            pltpu.VMEM((B_PER_W,), jnp.int32),
            pltpu.VMEM((L,), jnp.int32),
            pltpu.VMEM((L,), jnp.int32),
            pltpu.VMEM((L, 128), jnp.float32),
            pltpu.VMEM((L, 128), jnp.float32),
            pltpu.VMEM((L,), jnp.float32),
            pltpu.VMEM((CH,), jnp.int32),
            pltpu.VMEM((CH, 128), jnp.float32),
            pltpu.VMEM((CH, 128), jnp.float32),
            pltpu.VMEM((CH,), jnp.float32),
            pltpu.VMEM((CH,), jnp.float32),
            pltpu.VMEM((CH,), jnp.float32),
            pltpu.VMEM((2, CA, W), jnp.float32),
            pltpu.SemaphoreType.DMA((2,)),
            pltpu.SemaphoreType.DMA((2,)),
        ],
    )(aug_p, loc128, batch_idx)

    locs_b = jnp.stack([lx, ly], axis=1)
    aug_n = pl.pallas_call(
        _unpack_body,
        grid=(B // BN,),
        in_specs=[pl.BlockSpec((BN, W), lambda i: (i, 0))],
        out_specs=pl.BlockSpec((M1, NREP, BN), lambda i: (0, 0, i)),
        out_shape=jax.ShapeDtypeStruct((M1, NREP, B), jnp.float32),
    )(augout)
    aug_b = aug_n.transpose(1, 2, 0)
    return locs_b, aug_b, scales_b


# speedup vs baseline: 10.8803x; 1.3929x over previous
import functools

import jax
import jax.numpy as jnp
from jax import lax
from jax.experimental import pallas as pl
from jax.experimental.pallas import tpu as pltpu, tpu_sc as plsc

L = 16
NC = 2
NS = 16
NW = NC * NS
CA = 64
W = 512
BN = 4096


def _sqrt16(v):
    i = plsc.bitcast(v, jnp.int32)
    i = jnp.int32(0x5F3759DF) - (i >> 1)
    y = plsc.bitcast(i, jnp.float32)
    vh = v * jnp.float32(0.5)
    for _ in range(4):
        y = y * (jnp.float32(1.5) - vh * y * y)
    return v * y


def _build_body(aug_ref, locs_ref, cs_ref, out_ref):
    x = aug_ref[...].reshape(496, BN)
    xl = locs_ref[...]
    xc = cs_ref[0:1, :].astype(jnp.float32)
    pad = jnp.zeros((5, BN), jnp.float32)
    full = jnp.concatenate([x, xl, xc, pad], axis=0)
    out_ref[:, :504] = jnp.transpose(full)
    out_ref[:, 504:] = jnp.zeros((BN, 8), jnp.float32)


def _unpack_body(in_ref, out_ref):
    x = in_ref[...]
    out_ref[...] = jnp.transpose(x[:, :496]).reshape(31, 16, BN)


def _body(NREP, N, M1, B_PER_W,
          aug_hbm, bidx_hbm,
          augout, lxout, lyout, scout,
          idx_v, hidx_v, hnn_v, d16_v, nnwin_v,
          hrow_v, hnnrow_v, aug_v, nn_v2, lx_v, ly_v, sc_v, gsem, osem, nsem):
    wid = lax.axis_index("s") * NC + lax.axis_index("c")
    base = wid * B_PER_W
    lane = lax.iota(jnp.int32, L)
    cx = jnp.full((L,), 496, jnp.int32)
    cy = jnp.full((L,), 497, jnp.int32)
    cn = jnp.full((L,), 498, jnp.int32)

    pltpu.sync_copy(bidx_hbm.at[pl.ds(base, B_PER_W)], idx_v)

    hidx = jnp.where(lane == 1, 5, jnp.where(lane == 0, 1, lane + L + wid * L))
    hidx_v[...] = hidx
    pltpu.sync_copy(aug_hbm.at[hidx_v], hrow_v)
    hx = plsc.load_gather(hrow_v, [lane, cx])
    hy = plsc.load_gather(hrow_v, [lane, cy])
    hnn_v[...] = plsc.load_gather(hrow_v, [lane, cn]).astype(jnp.int32)
    pltpu.sync_copy(aug_hbm.at[hnn_v], hnnrow_v)
    dx = hx - plsc.load_gather(hnnrow_v, [lane, cx])
    dy = hy - plsc.load_gather(hnnrow_v, [lane, cy])
    d16_v[...] = dx * dx + dy * dy
    d16 = d16_v[...]
    d1 = d16[0]
    d5 = d16[1]

    def process(pb, t):
        rows_ref = aug_v.at[pb]
        for j in range(CA // L):
            rows = lane + j * L
            nnwin_v[pl.ds(j * L, L)] = plsc.load_gather(
                rows_ref, [rows, cn]).astype(jnp.int32)
        g = pltpu.make_async_copy(aug_hbm.at[nnwin_v], nn_v2, nsem)
        g.start()
        g.wait()
        for j in range(CA // L):
            rows = lane + j * L
            ax = plsc.load_gather(rows_ref, [rows, cx])
            ay = plsc.load_gather(rows_ref, [rows, cy])
            dx = ax - plsc.load_gather(nn_v2, [rows, cx])
            dy = ay - plsc.load_gather(nn_v2, [rows, cy])
            d = dx * dx + dy * dy
            sc = _sqrt16(d * d5) / d1
            o = t * CA + j * L
            bi = idx_v[pl.ds(o, L)]
            sc_v[pl.ds(o, L)] = jnp.where(bi == 0, jnp.float32(1.0), sc)
            lx_v[pl.ds(o, L)] = ax
            ly_v[pl.ds(o, L)] = ay

    nwin = B_PER_W // CA
    gathers = [None, None]
    writes = [None, None]
    for t in range(nwin):
        b = t % 2
        if writes[b] is not None:
            writes[b].wait()
        g = pltpu.make_async_copy(
            aug_hbm.at[idx_v.at[pl.ds(t * CA, CA)]], aug_v.at[b], gsem.at[b])
        g.start()
        gathers[b] = g
        if t >= 1:
            pb = 1 - b
            gathers[pb].wait()
            process(pb, t - 1)
            w = pltpu.make_async_copy(
                aug_v.at[pb], augout.at[pl.ds(base + (t - 1) * CA, CA)],
                osem.at[pb])
            w.start()
            writes[pb] = w
    lb = (nwin - 1) % 2
    gathers[lb].wait()
    process(lb, nwin - 1)
    w = pltpu.make_async_copy(
        aug_v.at[lb], augout.at[pl.ds(base + (nwin - 1) * CA, CA)], osem.at[lb])
    w.start()
    writes[lb] = w
    pltpu.sync_copy(sc_v, scout.at[pl.ds(base, B_PER_W)])
    pltpu.sync_copy(lx_v, lxout.at[pl.ds(base, B_PER_W)])
    pltpu.sync_copy(ly_v, lyout.at[pl.ds(base, B_PER_W)])
    writes[0].wait()
    writes[1].wait()


def kernel(locs, response, augmented_response, conditioning_sets, batch_idx):
    del response
    N, _ = locs.shape
    NREP, _, M1 = augmented_response.shape
    B = batch_idx.shape[0]
    B_PER_W = B // NW

    aug_t = augmented_response.transpose(2, 0, 1)
    locs_t = locs.T
    cs_t = conditioning_sets.T

    aug_p = pl.pallas_call(
        _build_body,
        grid=(pl.cdiv(N, BN),),
        in_specs=[
            pl.BlockSpec((M1, NREP, BN), lambda i: (0, 0, i)),
            pl.BlockSpec((2, BN), lambda i: (0, i)),
            pl.BlockSpec((8, BN), lambda i: (0, i)),
        ],
        out_specs=pl.BlockSpec((BN, W), lambda i: (i, 0)),
        out_shape=jax.ShapeDtypeStruct((N, W), jnp.float32),
    )(aug_t, locs_t, cs_t)

    mesh = plsc.VectorSubcoreMesh(core_axis_name="c", subcore_axis_name="s")
    body = functools.partial(_body, NREP, N, M1, B_PER_W)
    augout, lx, ly, scales_b = pl.kernel(
        body,
        mesh=mesh,
        compiler_params=pltpu.CompilerParams(needs_layout_passes=False),
        out_type=[
            jax.ShapeDtypeStruct((B, W), jnp.float32),
            jax.ShapeDtypeStruct((B,), jnp.float32),
            jax.ShapeDtypeStruct((B,), jnp.float32),
            jax.ShapeDtypeStruct((B,), jnp.float32),
        ],
        scratch_types=[
            pltpu.VMEM((B_PER_W,), jnp.int32),
            pltpu.VMEM((L,), jnp.int32),
            pltpu.VMEM((L,), jnp.int32),
            pltpu.VMEM((L,), jnp.float32),
            pltpu.VMEM((CA,), jnp.int32),
            pltpu.VMEM((L, W), jnp.float32),
            pltpu.VMEM((L, W), jnp.float32),
            pltpu.VMEM((2, CA, W), jnp.float32),
            pltpu.VMEM((CA, W), jnp.float32),
            pltpu.VMEM((B_PER_W,), jnp.float32),
            pltpu.VMEM((B_PER_W,), jnp.float32),
            pltpu.VMEM((B_PER_W,), jnp.float32),
            pltpu.SemaphoreType.DMA((2,)),
            pltpu.SemaphoreType.DMA((2,)),
            pltpu.SemaphoreType.DMA,
        ],
    )(aug_p, batch_idx)

    locs_b = jnp.stack([lx, ly], axis=1)
    aug_n = pl.pallas_call(
        _unpack_body,
        grid=(B // BN,),
        in_specs=[pl.BlockSpec((BN, W), lambda i: (i, 0))],
        out_specs=pl.BlockSpec((M1, NREP, BN), lambda i: (0, 0, i)),
        out_shape=jax.ShapeDtypeStruct((M1, NREP, B), jnp.float32),
    )(augout)
    aug_b = aug_n.transpose(1, 2, 0)
    return locs_b, aug_b, scales_b
